# baseline (speedup 1.0000x reference)
import functools

import jax
import jax.numpy as jnp
from jax import lax
from jax.experimental import pallas as pl
from jax.experimental.pallas import tpu as pltpu
from jax.experimental.pallas import tpu_sc as plsc

BATCH = 4096
SEQ = 200
D = 64
NC, NS = 2, 16
NW = NC * NS
GROUP = 128
BBLK = 512
NGRP = BBLK // GROUP
NQ = BATCH // BBLK
NUNITS = SEQ * NQ
UPW = NUNITS // NW
NHALF = UPW // 2
LANES = 16


def _emb_body(idx_hbm, table_hbm, out_hbm, idxbuf, rowbuf, rowt, gsem0,
              gsem1, wsem):
    wid = lax.axis_index("s") * NC + lax.axis_index("c")
    ubase = wid * UPW
    gsem = (gsem0, gsem1)

    def load_idx(u, slot):
        s, q = u // NQ, u % NQ
        pltpu.sync_copy(idx_hbm.at[s, pl.ds(NGRP * q, NGRP)], idxbuf.at[slot])

    def fire_gathers(slot):
        for j in range(NGRP):
            pltpu.async_copy(table_hbm.at[idxbuf.at[slot, j]],
                             rowbuf.at[slot, pl.ds(j * GROUP, GROUP)],
                             gsem[slot])

    def transpose_chunk(slot, j):
        block = rowbuf.at[slot]
        eiota = lax.iota(jnp.int32, LANES)
        evecs = [eiota + (c * LANES) for c in range(D // LANES)]

        def b_body(b, b_vec):
            vals = [block[b, pl.ds(c * LANES, LANES)]
                    for c in range(D // LANES)]
            for c in range(D // LANES):
                plsc.store_scatter(rowt, [evecs[c], b_vec], vals[c])
            return b_vec + 1

        lax.fori_loop(j * GROUP, (j + 1) * GROUP, b_body,
                      jnp.full((LANES,), j * GROUP, jnp.int32), unroll=8)

    def drain_and_transpose(slot):
        for j in range(NGRP):
            pltpu.make_async_copy(table_hbm.at[idxbuf.at[slot, j]],
                                  rowbuf.at[slot, pl.ds(j * GROUP, GROUP)],
                                  gsem[slot]).wait()
            transpose_chunk(slot, j)

    def fire_write(u):
        s, q = u // NQ, u % NQ
        pltpu.async_copy(rowt.at[slice(None), pl.ds(0, BBLK)],
                         out_hbm.at[s, slice(None), pl.ds(BBLK * q, BBLK)],
                         wsem)

    def wait_write(u):
        s, q = u // NQ, u % NQ
        pltpu.make_async_copy(rowt.at[slice(None), pl.ds(0, BBLK)],
                              out_hbm.at[s, slice(None), pl.ds(BBLK * q, BBLK)],
                              wsem).wait()

    load_idx(ubase, 0)
    fire_gathers(0)

    def body(i, carry):
        a = ubase + 2 * i
        b = a + 1

        load_idx(b, 1)
        fire_gathers(1)

        @pl.when(i > 0)
        def _():
            wait_write(a - 1)
        drain_and_transpose(0)
        fire_write(a)

        @pl.when(i + 1 < NHALF)
        def _():
            load_idx(a + 2, 0)
            fire_gathers(0)

        wait_write(a)
        drain_and_transpose(1)
        fire_write(b)
        return carry

    lax.fori_loop(0, NHALF, body, 0)

    wait_write(ubase + UPW - 1)


@jax.jit
def _emb_call(idx3, table):
    mesh = plsc.VectorSubcoreMesh(core_axis_name="c", subcore_axis_name="s")
    run = functools.partial(
        pl.kernel,
        mesh=mesh,
        compiler_params=pltpu.CompilerParams(
            use_tc_tiling_on_sc=False, needs_layout_passes=False),
        out_type=jax.ShapeDtypeStruct((SEQ, D, BATCH), jnp.float32),
        scratch_types=[
            pltpu.VMEM((2, NGRP, GROUP), jnp.int32),
            pltpu.VMEM((2, BBLK, D), jnp.float32),
            pltpu.VMEM((D, BBLK + 1), jnp.float32),
            pltpu.SemaphoreType.DMA,
            pltpu.SemaphoreType.DMA,
            pltpu.SemaphoreType.DMA,
        ],
    )(_emb_body)
    return run(idx3, table)


def kernel(indices, table):
    idx3 = indices.astype(jnp.int32).T.reshape(SEQ, BATCH // GROUP, GROUP)
    out_t = _emb_call(idx3, table)
    return jnp.transpose(out_t, (2, 0, 1))
